# baseline (speedup 1.0000x reference)
import functools

import jax
import jax.numpy as jnp
from jax import lax
from jax.experimental import pallas as pl
from jax.experimental.pallas import tpu as pltpu
from jax.experimental.pallas import tpu_sc as plsc

_B, _S, _D = 4, 8192, 1024
_NC, _NS = 2, 16
_NW = _NC * _NS
_RPW = _S // _NW
_CH = 32
_NCHUNK = _RPW // _CH
_NBUF = 3


def _sc_body(w_hbm, o_hbm, bufs, rsems, wsems):
    c = lax.axis_index("c")
    s = lax.axis_index("s")
    wid = s * _NC + c
    base = wid * _RPW

    def start_read(i):
        return pltpu.async_copy(
            w_hbm.at[pl.ds(base + i * _CH, _CH)], bufs[i % _NBUF],
            rsems[i % _NBUF])

    reads = {0: start_read(0)}
    writes = {}
    for i in range(_NCHUNK):
        reads.pop(i).wait()
        writes[i] = [
            pltpu.async_copy(
                bufs[i % _NBUF], o_hbm.at[b, pl.ds(base + i * _CH, _CH)],
                wsems[i % _NBUF])
            for b in range(_B)
        ]
        if i - 2 >= 0:
            for h in writes.pop(i - 2):
                h.wait()
        if i + 1 < _NCHUNK:
            reads[i + 1] = start_read(i + 1)
    for i in (_NCHUNK - 2, _NCHUNK - 1):
        for h in writes.pop(i, []):
            h.wait()


@functools.partial(
    pl.kernel,
    out_type=jax.ShapeDtypeStruct((_B, _S, _D), jnp.float32),
    mesh=plsc.VectorSubcoreMesh(core_axis_name="c", subcore_axis_name="s"),
    scratch_types=[
        pltpu.VMEM((_CH, _D), jnp.float32),
        pltpu.VMEM((_CH, _D), jnp.float32),
        pltpu.VMEM((_CH, _D), jnp.float32),
        pltpu.SemaphoreType.DMA,
        pltpu.SemaphoreType.DMA,
        pltpu.SemaphoreType.DMA,
        pltpu.SemaphoreType.DMA,
        pltpu.SemaphoreType.DMA,
        pltpu.SemaphoreType.DMA,
    ],
)
def _sc_broadcast_copy(w_hbm, o_hbm, b0, b1, b2, r0, r1, r2, w0, w1, w2):
    _sc_body(w_hbm, o_hbm, (b0, b1, b2), (r0, r1, r2), (w0, w1, w2))


def kernel(x, weight):
    return _sc_broadcast_copy(weight)


# speedup vs baseline: 1.0534x; 1.0534x over previous
import functools

import jax
import jax.numpy as jnp
from jax import lax
from jax.experimental import pallas as pl
from jax.experimental.pallas import tpu as pltpu
from jax.experimental.pallas import tpu_sc as plsc

_B, _S, _D = 4, 8192, 1024
_S_SC = 4096
_S_TC = _S - _S_SC
_NC, _NS = 2, 16
_NW = _NC * _NS
_RPW = _S_SC // _NW
_CH = 32
_NCHUNK = _RPW // _CH
_NBUF = 3
_BS_TC = 1024


def _sc_body(w_hbm, o_hbm, bufs, rsems, wsems):
    c = lax.axis_index("c")
    s = lax.axis_index("s")
    wid = s * _NC + c
    base = _S_TC + wid * _RPW

    def start_read(i):
        return pltpu.async_copy(
            w_hbm.at[pl.ds(base + i * _CH, _CH)], bufs[i % _NBUF],
            rsems[i % _NBUF])

    reads = {0: start_read(0)}
    writes = {}
    for i in range(_NCHUNK):
        reads.pop(i).wait()
        writes[i] = [
            pltpu.async_copy(
                bufs[i % _NBUF], o_hbm.at[b, pl.ds(base + i * _CH, _CH)],
                wsems[i % _NBUF])
            for b in range(_B)
        ]
        if i - 2 >= 0:
            for h in writes.pop(i - 2):
                h.wait()
        if i + 1 < _NCHUNK:
            reads[i + 1] = start_read(i + 1)
    for i in (_NCHUNK - 2, _NCHUNK - 1):
        for h in writes.pop(i, []):
            h.wait()


@functools.partial(
    pl.kernel,
    out_type=jax.ShapeDtypeStruct((_B, _S, _D), jnp.float32),
    mesh=plsc.VectorSubcoreMesh(core_axis_name="c", subcore_axis_name="s"),
    scratch_types=[
        pltpu.VMEM((_CH, _D), jnp.float32),
        pltpu.VMEM((_CH, _D), jnp.float32),
        pltpu.VMEM((_CH, _D), jnp.float32),
        pltpu.SemaphoreType.DMA,
        pltpu.SemaphoreType.DMA,
        pltpu.SemaphoreType.DMA,
        pltpu.SemaphoreType.DMA,
        pltpu.SemaphoreType.DMA,
        pltpu.SemaphoreType.DMA,
    ],
)
def _sc_tail_copy(w_hbm, o_hbm, b0, b1, b2, r0, r1, r2, w0, w1, w2):
    _sc_body(w_hbm, o_hbm, (b0, b1, b2), (r0, r1, r2), (w0, w1, w2))


def _tc_head_copy(w_ref, alias_ref, o_ref):
    del alias_ref
    o_ref[...] = jnp.broadcast_to(w_ref[...][None], o_ref.shape)


def kernel(x, weight):
    partial = _sc_tail_copy(weight)
    return pl.pallas_call(
        _tc_head_copy,
        grid=(_S_TC // _BS_TC,),
        in_specs=[
            pl.BlockSpec((_BS_TC, _D), lambda s: (s, 0)),
            pl.BlockSpec(memory_space=pl.ANY),
        ],
        out_specs=pl.BlockSpec((_B, _BS_TC, _D), lambda s: (0, s, 0)),
        out_shape=jax.ShapeDtypeStruct((_B, _S, _D), weight.dtype),
        input_output_aliases={1: 0},
    )(weight, partial)


# speedup vs baseline: 1.1029x; 1.0470x over previous
import functools

import jax
import jax.numpy as jnp
from jax import lax
from jax.experimental import pallas as pl
from jax.experimental.pallas import tpu as pltpu
from jax.experimental.pallas import tpu_sc as plsc

_B, _S, _D = 4, 8192, 1024
_S_SC = 3072
_S_TC = _S - _S_SC
_NC, _NS = 2, 16
_NW = _NC * _NS
_RPW = _S_SC // _NW
_CH = 32
_NCHUNK = _RPW // _CH
_NBUF = 3
_BS_TC = 1024


def _alloc_body(o_ref):
    pass


def _alloc_out():
    return pl.pallas_call(
        _alloc_body,
        out_specs=pl.BlockSpec(memory_space=pl.ANY),
        out_shape=jax.ShapeDtypeStruct((_B, _S, _D), jnp.float32),
    )()


def _sc_body(w_hbm, buf_hbm, bufs, rsems, wsems):
    c = lax.axis_index("c")
    s = lax.axis_index("s")
    wid = s * _NC + c
    base = _S_TC + wid * _RPW

    def start_read(i):
        return pltpu.async_copy(
            w_hbm.at[pl.ds(base + i * _CH, _CH)], bufs[i % _NBUF],
            rsems[i % _NBUF])

    reads = {0: start_read(0)}
    writes = {}
    for i in range(_NCHUNK):
        reads.pop(i).wait()
        writes[i] = [
            pltpu.async_copy(
                bufs[i % _NBUF], buf_hbm.at[b, pl.ds(base + i * _CH, _CH)],
                wsems[i % _NBUF])
            for b in range(_B)
        ]
        if i - 2 >= 0:
            for h in writes.pop(i - 2):
                h.wait()
        if i + 1 < _NCHUNK:
            reads[i + 1] = start_read(i + 1)
    for i in (_NCHUNK - 2, _NCHUNK - 1):
        for h in writes.pop(i, []):
            h.wait()


@functools.partial(
    pl.kernel,
    out_type=jax.ShapeDtypeStruct((16,), jnp.float32),
    mesh=plsc.VectorSubcoreMesh(core_axis_name="c", subcore_axis_name="s"),
    scratch_types=[
        pltpu.VMEM((_CH, _D), jnp.float32),
        pltpu.VMEM((_CH, _D), jnp.float32),
        pltpu.VMEM((_CH, _D), jnp.float32),
        pltpu.SemaphoreType.DMA,
        pltpu.SemaphoreType.DMA,
        pltpu.SemaphoreType.DMA,
        pltpu.SemaphoreType.DMA,
        pltpu.SemaphoreType.DMA,
        pltpu.SemaphoreType.DMA,
    ],
)
def _sc_tail_writer(w_hbm, buf_hbm, dummy_out, b0, b1, b2,
                    r0, r1, r2, w0, w1, w2):
    del dummy_out
    _sc_body(w_hbm, buf_hbm, (b0, b1, b2), (r0, r1, r2), (w0, w1, w2))


def _tc_head_writer_body(w_ref, buf_ref, o_ref, sem):
    s = pl.program_id(0)
    o_ref[...] = jnp.zeros_like(o_ref)
    copies = [
        pltpu.make_async_copy(
            w_ref, buf_ref.at[b, pl.ds(s * _BS_TC, _BS_TC)], sem)
        for b in range(_B)
    ]
    for cp in copies:
        cp.start()
    for cp in copies:
        cp.wait()


def _tc_head_writer(weight, buf):
    return pl.pallas_call(
        _tc_head_writer_body,
        grid=(_S_TC // _BS_TC,),
        in_specs=[
            pl.BlockSpec((_BS_TC, _D), lambda s: (s, 0)),
            pl.BlockSpec(memory_space=pl.ANY),
        ],
        out_specs=pl.BlockSpec((8, 128), lambda s: (0, 0)),
        out_shape=jax.ShapeDtypeStruct((8, 128), jnp.float32),
        scratch_shapes=[pltpu.SemaphoreType.DMA],
    )(weight, buf)


def _finish_body(buf_ref, d_sc_ref, d_tc_ref, o_ref):
    pass


def _finish(buf, d_sc, d_tc):
    return pl.pallas_call(
        _finish_body,
        in_specs=[
            pl.BlockSpec(memory_space=pl.ANY),
            pl.BlockSpec(memory_space=pl.ANY),
            pl.BlockSpec(memory_space=pl.ANY),
        ],
        out_specs=pl.BlockSpec(memory_space=pl.ANY),
        out_shape=jax.ShapeDtypeStruct((_B, _S, _D), jnp.float32),
        input_output_aliases={0: 0},
    )(buf, d_sc, d_tc)


def kernel(x, weight):
    buf = _alloc_out()
    d_sc = _sc_tail_writer(weight, buf)
    d_tc = _tc_head_writer(weight, buf)
    return _finish(buf, d_sc, d_tc)
